# Initial kernel scaffold; baseline (speedup 1.0000x reference)
import jax
import jax.numpy as jnp
from jax.experimental import pallas as pl


def kernel(z, edge_index, W0, b0, W1, b1, W2, b2, Wr1, br1, Wr2, br2, Wt1, bt1, Wt2, bt2):
    raise NotImplementedError("write your pallas kernel here")


# speedup vs baseline: 4.0003x; 4.0003x over previous
import functools

import jax
import jax.numpy as jnp
from jax import lax
from jax.experimental import pallas as pl
from jax.experimental.pallas import tpu as pltpu
from jax.experimental.pallas import tpu_sc as plsc

N = 50000
E = 800000
EP = 819200
CH = 128
NPAD = 50176
STRIPE = 3136
ZCH = 448
NC = 2
NS = 16
B = 1000
GRID = N // B
RADIUS = 1.0

_f32 = jnp.float32
_i32 = jnp.int32


def _mesh():
    return plsc.VectorSubcoreMesh(core_axis_name="c", subcore_axis_name="s")


def _deg_body(dst_hbm, zeros_hbm, ones_hbm, out_hbm, ones_v, idx_v, acc):
    c = lax.axis_index("c")
    s = lax.axis_index("s")
    for k in range(STRIPE // ZCH):
        pltpu.sync_copy(zeros_hbm, acc.at[pl.ds(s * STRIPE + k * ZCH, ZCH)])
    plsc.subcore_barrier()
    pltpu.sync_copy(ones_hbm, ones_v)
    base0 = (c * NS + s) * (EP // (NC * NS))

    def chunk(k, carry):
        b = base0 + k * CH
        pltpu.sync_copy(dst_hbm.at[pl.ds(b, CH)], idx_v)
        pltpu.sync_copy(ones_v, acc.at[idx_v], add=True)
        return carry

    lax.fori_loop(0, EP // (NC * NS * CH), chunk, 0)
    plsc.subcore_barrier()
    pltpu.sync_copy(acc.at[pl.ds(s * STRIPE, STRIPE)],
                    out_hbm.at[pl.ds(c * NPAD + s * STRIPE, STRIPE)])


def _deg_call(dst_seg):
    f = pl.kernel(
        _deg_body,
        out_type=jax.ShapeDtypeStruct((2 * NPAD, 16), _f32),
        mesh=_mesh(),
        scratch_types=[
            pltpu.VMEM((CH, 16), _f32),
            pltpu.VMEM((CH,), _i32),
            pltpu.VMEM_SHARED((NPAD, 16), _f32),
        ],
        compiler_params=pltpu.CompilerParams(use_tc_tiling_on_sc=False),
        name="sc_degree",
    )
    return f(dst_seg, jnp.zeros((ZCH, 16), _f32), jnp.ones((CH, 16), _f32))


def _seg_body(taba_hbm, tabb_hbm, src_hbm, dst_hbm, zeros_hbm, out_hbm,
              sidx, didx, rows, acc, sem):
    c = lax.axis_index("c")
    s = lax.axis_index("s")
    for k in range(STRIPE // ZCH):
        pltpu.sync_copy(zeros_hbm, acc.at[pl.ds(s * STRIPE + k * ZCH, ZCH)])
    plsc.subcore_barrier()
    base0 = s * (EP // NS)

    def chunk(k, carry):
        b = base0 + k * CH
        pltpu.sync_copy(src_hbm.at[pl.ds(b, CH)], sidx)
        pltpu.sync_copy(dst_hbm.at[pl.ds(b, CH)], didx)

        @pl.when(c == 0)
        def _():
            pltpu.async_copy(taba_hbm.at[sidx], rows, sem).wait()

        @pl.when(c == 1)
        def _():
            pltpu.async_copy(tabb_hbm.at[sidx], rows, sem).wait()

        pltpu.sync_copy(rows, acc.at[didx], add=True)
        return carry

    lax.fori_loop(0, EP // (NS * CH), chunk, 0)
    plsc.subcore_barrier()
    pltpu.sync_copy(acc.at[pl.ds(s * STRIPE, STRIPE)],
                    out_hbm.at[pl.ds(c * NPAD + s * STRIPE, STRIPE)])


def _seg_call(tab2, src_p, dst_seg, w2):
    f = pl.kernel(
        _seg_body,
        out_type=jax.ShapeDtypeStruct((2 * NPAD, w2), _f32),
        mesh=_mesh(),
        scratch_types=[
            pltpu.VMEM((CH,), _i32),
            pltpu.VMEM((CH,), _i32),
            pltpu.VMEM((CH, w2), _f32),
            pltpu.VMEM_SHARED((NPAD, w2), _f32),
            pltpu.SemaphoreType.DMA,
        ],
        compiler_params=pltpu.CompilerParams(use_tc_tiling_on_sc=False),
        name=f"sc_segsum{w2}",
    )
    return f(tab2[0], tab2[1], src_p, dst_seg, jnp.zeros((ZCH, w2), _f32))


def _edge_body(u_hbm, v_hbm, zs_hbm, zd_hbm, src_hbm, dst_hbm, wb_hbm,
               r_hbm, t_hbm, in_hbm,
               sidx, didx, p, zs, zd, wb, rbuf, tbuf, ibuf, sem):
    c = lax.axis_index("c")
    s = lax.axis_index("s")
    pltpu.sync_copy(wb_hbm, wb)
    base0 = (c * NS + s) * (EP // (NC * NS))
    lanes = lax.iota(_i32, 16)

    def col(j):
        return jnp.full((16,), j, _i32)

    def chunk(k, carry):
        b = base0 + k * CH
        pltpu.sync_copy(src_hbm.at[pl.ds(b, CH)], sidx)
        pltpu.sync_copy(dst_hbm.at[pl.ds(b, CH)], didx)
        pltpu.async_copy(u_hbm.at[sidx], p, sem).wait()
        pltpu.async_copy(v_hbm.at[didx], p, sem, add=True).wait()
        pltpu.async_copy(zs_hbm.at[sidx], zs, sem).wait()
        pltpu.async_copy(zd_hbm.at[didx], zd, sem).wait()

        def group(g, carry2):
            row = g * 16 + lanes
            accr = jnp.zeros((16,), _f32)
            acct = jnp.zeros((16,), _f32)
            acci = jnp.zeros((16,), _f32)
            for j in range(64):
                x = plsc.load_gather(p, [row, col(j)])
                accr = accr + jnp.maximum(x, 0.2 * x) * wb[j]
            for j in range(64):
                x = plsc.load_gather(p, [row, col(64 + j)])
                acct = acct + jnp.maximum(x, 0.2 * x) * wb[64 + j]
            for j in range(17):
                a = plsc.load_gather(zs, [row, col(j)])
                bb = plsc.load_gather(zd, [row, col(j)])
                acci = acci + a * bb
            rbuf[pl.ds(g * 16, 16)] = accr
            tbuf[pl.ds(g * 16, 16)] = acct
            ibuf[pl.ds(g * 16, 16)] = acci
            return carry2

        lax.fori_loop(0, CH // 16, group, 0)
        pltpu.sync_copy(rbuf, r_hbm.at[pl.ds(b, CH)])
        pltpu.sync_copy(tbuf, t_hbm.at[pl.ds(b, CH)])
        pltpu.sync_copy(ibuf, in_hbm.at[pl.ds(b, CH)])
        return carry

    lax.fori_loop(0, EP // (NC * NS * CH), chunk, 0)


def _edge_call(U, V, Zs, Zd, src_p, dst_e, wb):
    f = pl.kernel(
        _edge_body,
        out_type=[jax.ShapeDtypeStruct((EP,), _f32)] * 3,
        mesh=_mesh(),
        scratch_types=[
            pltpu.VMEM((CH,), _i32),
            pltpu.VMEM((CH,), _i32),
            pltpu.VMEM((CH, 128), _f32),
            pltpu.VMEM((CH, 32), _f32),
            pltpu.VMEM((CH, 32), _f32),
            pltpu.VMEM((128, 16), _f32),
            pltpu.VMEM((CH,), _f32),
            pltpu.VMEM((CH,), _f32),
            pltpu.VMEM((CH,), _f32),
            pltpu.SemaphoreType.DMA,
        ],
        compiler_params=pltpu.CompilerParams(
            use_tc_tiling_on_sc=False, needs_layout_passes=False),
        name="sc_edge",
    )
    return f(U, V, Zs, Zd, src_p, dst_e, wb)


def _prep_body(z_ref, d0_ref, d1_ref, w0_ref, yp_ref, dinv_ref, zs_ref, zd_ref):
    z = z_ref[...]
    deg = d0_ref[0][:, :1] + d1_ref[0][:, :1] + 1.0
    dinv = lax.rsqrt(deg)
    dinv_ref[...] = dinv
    x0 = z[:, :1]
    alpha = jnp.maximum(x0 / RADIUS, 1.0 + 1e-7)
    acosh = jnp.log(alpha + jnp.sqrt(alpha * alpha - 1.0))
    coef = acosh / jnp.sqrt(alpha * alpha - 1.0)
    zmu = coef * jnp.concatenate([x0 - alpha * RADIUS, z[:, 1:]], axis=1)
    y = (zmu @ w0_ref[...]) * dinv
    yp_ref[0] = y[:, :32]
    yp_ref[1] = y[:, 32:]
    pad = jnp.zeros((z.shape[0], 15), _f32)
    zs_ref[...] = jnp.concatenate([-x0, z[:, 1:], pad], axis=1)
    zd_ref[...] = jnp.concatenate([x0, z[:, 1:], pad], axis=1)


def _prep_call(z, deg2, W0):
    return pl.pallas_call(
        _prep_body,
        grid=(GRID,),
        in_specs=[
            pl.BlockSpec((B, 17), lambda i: (i, 0)),
            pl.BlockSpec((1, B, 16), lambda i: (0, i, 0)),
            pl.BlockSpec((1, B, 16), lambda i: (1, i, 0)),
            pl.BlockSpec((17, 64), lambda i: (0, 0)),
        ],
        out_specs=[
            pl.BlockSpec((2, B, 32), lambda i: (0, i, 0)),
            pl.BlockSpec((B, 1), lambda i: (i, 0)),
            pl.BlockSpec((B, 32), lambda i: (i, 0)),
            pl.BlockSpec((B, 32), lambda i: (i, 0)),
        ],
        out_shape=[
            jax.ShapeDtypeStruct((2, N, 32), _f32),
            jax.ShapeDtypeStruct((N, 1), _f32),
            jax.ShapeDtypeStruct((N, 32), _f32),
            jax.ShapeDtypeStruct((N, 32), _f32),
        ],
        name="tc_prep",
    )(z, deg2, deg2, W0)


def _layer_body(acc_ref, yp_ref, dinv_ref, b_ref, w_ref, out_ref, *, relu, wo2):
    y = jnp.concatenate([acc_ref[0] + yp_ref[0], acc_ref[1] + yp_ref[1]], axis=1)
    h = dinv_ref[...] * y + b_ref[...]
    if relu:
        h = jnp.maximum(h, 0.0)
    yn = (h @ w_ref[...]) * dinv_ref[...]
    out_ref[0] = yn[:, :wo2]
    out_ref[1] = yn[:, wo2:]


def _layer_call(acc, yp, dinv, bias, W, relu):
    wi2 = yp.shape[2]
    wo2 = W.shape[1] // 2
    body = functools.partial(_layer_body, relu=relu, wo2=wo2)
    return pl.pallas_call(
        body,
        grid=(GRID,),
        in_specs=[
            pl.BlockSpec((2, B, wi2), lambda i: (0, i, 0)),
            pl.BlockSpec((2, B, wi2), lambda i: (0, i, 0)),
            pl.BlockSpec((B, 1), lambda i: (i, 0)),
            pl.BlockSpec((2 * wi2,), lambda i: (0,)),
            pl.BlockSpec(W.shape, lambda i: (0, 0)),
        ],
        out_specs=pl.BlockSpec((2, B, wo2), lambda i: (0, i, 0)),
        out_shape=jax.ShapeDtypeStruct((2, N, wo2), _f32),
        name=f"tc_layer{W.shape[1]}",
    )(acc, yp, dinv, bias, W)


def _final_node_body(acc_ref, yp_ref, dinv_ref, b_ref, wu_ref, wv_ref, bv_ref,
                     u_ref, v_ref):
    y = jnp.concatenate([acc_ref[0] + yp_ref[0], acc_ref[1] + yp_ref[1]], axis=1)
    g = dinv_ref[...] * y + b_ref[...]
    u_ref[...] = g @ wu_ref[...]
    v_ref[...] = g @ wv_ref[...] + bv_ref[...]


def _final_node_call(acc, yp, dinv, b2, WU, WV, bV):
    return pl.pallas_call(
        _final_node_body,
        grid=(GRID,),
        in_specs=[
            pl.BlockSpec((2, B, 16), lambda i: (0, i, 0)),
            pl.BlockSpec((2, B, 16), lambda i: (0, i, 0)),
            pl.BlockSpec((B, 1), lambda i: (i, 0)),
            pl.BlockSpec((32,), lambda i: (0,)),
            pl.BlockSpec((32, 128), lambda i: (0, 0)),
            pl.BlockSpec((32, 128), lambda i: (0, 0)),
            pl.BlockSpec((128,), lambda i: (0,)),
        ],
        out_specs=[
            pl.BlockSpec((B, 128), lambda i: (i, 0)),
            pl.BlockSpec((B, 128), lambda i: (i, 0)),
        ],
        out_shape=[
            jax.ShapeDtypeStruct((N, 128), _f32),
            jax.ShapeDtypeStruct((N, 128), _f32),
        ],
        name="tc_final_node",
    )(acc, yp, dinv, b2, WU, WV, bV)


def _combine_body(in_ref, r_ref, t_ref, br_ref, bt_ref, out_ref):
    arg = jnp.maximum(-in_ref[...] / (RADIUS * RADIUS), 1.0 + 1e-7)
    dist = -RADIUS * jnp.log(arg + jnp.sqrt(arg * arg - 1.0))
    r = r_ref[...] + br_ref[0]
    t = t_ref[...] + bt_ref[0]
    x = (dist - r) / t
    out_ref[...] = 1.0 / (1.0 + jnp.exp(-x))


def _combine_call(inner, rpre, tpre, br2, bt2):
    rows = EP // 128
    blk = 64
    return pl.pallas_call(
        _combine_body,
        grid=(rows // blk,),
        in_specs=[
            pl.BlockSpec((blk, 128), lambda i: (i, 0)),
            pl.BlockSpec((blk, 128), lambda i: (i, 0)),
            pl.BlockSpec((blk, 128), lambda i: (i, 0)),
            pl.BlockSpec(memory_space=pltpu.SMEM),
            pl.BlockSpec(memory_space=pltpu.SMEM),
        ],
        out_specs=pl.BlockSpec((blk, 128), lambda i: (i, 0)),
        out_shape=jax.ShapeDtypeStruct((rows, 128), _f32),
        name="tc_combine",
    )(inner.reshape(rows, 128), rpre.reshape(rows, 128),
      tpre.reshape(rows, 128), br2, bt2)


def kernel(z, edge_index, W0, b0, W1, b1, W2, b2,
           Wr1, br1, Wr2, br2, Wt1, bt1, Wt2, bt2):
    src = edge_index[0]
    dst = edge_index[1]
    padn = EP - E
    zero_pad = jnp.zeros((padn,), _i32)
    src_p = jnp.concatenate([src, zero_pad])
    dst_e = jnp.concatenate([dst, zero_pad])
    dst_seg = jnp.concatenate([dst, jnp.full((padn,), N, _i32)])

    deg2 = _deg_call(dst_seg).reshape(2, NPAD, 16)
    yp0, dinv, Zs, Zd = _prep_call(z, deg2, W0)

    acc0 = _seg_call(yp0, src_p, dst_seg, 32).reshape(2, NPAD, 32)
    yp1 = _layer_call(acc0, yp0, dinv, b0, W1, relu=True)
    acc1 = _seg_call(yp1, src_p, dst_seg, 32).reshape(2, NPAD, 32)
    yp2 = _layer_call(acc1, yp1, dinv, b1, W2, relu=True)
    acc2 = _seg_call(yp2, src_p, dst_seg, 16).reshape(2, NPAD, 16)

    WU = jnp.concatenate([Wr1[:32], Wt1[:32]], axis=1)
    WV = jnp.concatenate([Wr1[32:], Wt1[32:]], axis=1)
    bV = jnp.concatenate([br1, bt1])
    U, V = _final_node_call(acc2, yp2, dinv, b2, WU, WV, bV)

    wb = jnp.concatenate([jnp.broadcast_to(Wr2, (64, 16)),
                          jnp.broadcast_to(Wt2, (64, 16))], axis=0)
    rpre, tpre, inner = _edge_call(U, V, Zs, Zd, src_p, dst_e, wb)
    probs = _combine_call(inner, rpre, tpre, br2, bt2)
    return probs.reshape(EP)[:E]


# speedup vs baseline: 5.3122x; 1.3279x over previous
import functools

import jax
import jax.numpy as jnp
from jax import lax
from jax.experimental import pallas as pl
from jax.experimental.pallas import tpu as pltpu
from jax.experimental.pallas import tpu_sc as plsc

N = 50000
E = 800000
EP = 819200
CH = 128
NPAD = 50176
STRIPE = 3136
ZCH = 448
NC = 2
NS = 16
SUB = 4
B = 1000
GRID = N // B
RADIUS = 1.0

_f32 = jnp.float32
_i32 = jnp.int32


def _mesh():
    return plsc.VectorSubcoreMesh(core_axis_name="c", subcore_axis_name="s")


def _deg_body(dst_hbm, zeros_hbm, ones_hbm, out_hbm, ones_v, idx_v, acc):
    c = lax.axis_index("c")
    s = lax.axis_index("s")
    for k in range(STRIPE // ZCH):
        pltpu.sync_copy(zeros_hbm, acc.at[pl.ds(s * STRIPE + k * ZCH, ZCH)])
    plsc.subcore_barrier()
    pltpu.sync_copy(ones_hbm, ones_v)
    base0 = (c * NS + s) * (EP // (NC * NS))

    def chunk(k, carry):
        b = base0 + k * CH
        pltpu.sync_copy(dst_hbm.at[pl.ds(b, CH)], idx_v)
        pltpu.sync_copy(ones_v, acc.at[idx_v], add=True)
        return carry

    lax.fori_loop(0, EP // (NC * NS * CH), chunk, 0)
    plsc.subcore_barrier()
    pltpu.sync_copy(acc.at[pl.ds(s * STRIPE, STRIPE)],
                    out_hbm.at[pl.ds(c * NPAD + s * STRIPE, STRIPE)])


def _deg_call(dst_seg):
    f = pl.kernel(
        _deg_body,
        out_type=jax.ShapeDtypeStruct((2 * NPAD, 16), _f32),
        mesh=_mesh(),
        scratch_types=[
            pltpu.VMEM((CH, 16), _f32),
            pltpu.VMEM((CH,), _i32),
            pltpu.VMEM_SHARED((NPAD, 16), _f32),
        ],
        compiler_params=pltpu.CompilerParams(use_tc_tiling_on_sc=False),
        name="sc_degree",
    )
    return f(dst_seg, jnp.zeros((ZCH, 16), _f32), jnp.ones((CH, 16), _f32))


def _seg_body(taba_hbm, tabb_hbm, src_hbm, dst_hbm, zeros_hbm, out_hbm,
              sidx, didx, rows, acc, sem):
    c = lax.axis_index("c")
    s = lax.axis_index("s")
    for k in range(STRIPE // ZCH):
        pltpu.sync_copy(zeros_hbm, acc.at[pl.ds(s * STRIPE + k * ZCH, ZCH)])
    plsc.subcore_barrier()
    base0 = s * (EP // (NS * CH))

    def chunk(k, carry):
        r0 = base0 + k * SUB
        pltpu.sync_copy(src_hbm.at[pl.ds(r0, SUB)], sidx)
        pltpu.sync_copy(dst_hbm.at[pl.ds(r0, SUB)], didx)

        @pl.when(c == 0)
        def _():
            ds = [pltpu.async_copy(taba_hbm.at[sidx.at[i]],
                                   rows.at[pl.ds(i * CH, CH)], sem)
                  for i in range(SUB)]
            for d in ds:
                d.wait()

        @pl.when(c == 1)
        def _():
            ds = [pltpu.async_copy(tabb_hbm.at[sidx.at[i]],
                                   rows.at[pl.ds(i * CH, CH)], sem)
                  for i in range(SUB)]
            for d in ds:
                d.wait()

        for i in range(SUB):
            pltpu.sync_copy(rows.at[pl.ds(i * CH, CH)], acc.at[didx.at[i]],
                            add=True)
        return carry

    lax.fori_loop(0, EP // (NS * CH * SUB), chunk, 0)
    plsc.subcore_barrier()
    pltpu.sync_copy(acc.at[pl.ds(s * STRIPE, STRIPE)],
                    out_hbm.at[pl.ds(c * NPAD + s * STRIPE, STRIPE)])


def _seg_call(tab2, src2d, dst2d, w2):
    f = pl.kernel(
        _seg_body,
        out_type=jax.ShapeDtypeStruct((2 * NPAD, w2), _f32),
        mesh=_mesh(),
        scratch_types=[
            pltpu.VMEM((SUB, CH), _i32),
            pltpu.VMEM((SUB, CH), _i32),
            pltpu.VMEM((SUB * CH, w2), _f32),
            pltpu.VMEM_SHARED((NPAD, w2), _f32),
            pltpu.SemaphoreType.DMA,
        ],
        compiler_params=pltpu.CompilerParams(use_tc_tiling_on_sc=False),
        name=f"sc_segsum{w2}",
    )
    return f(tab2[0], tab2[1], src2d, dst2d, jnp.zeros((ZCH, w2), _f32))


def _edge_body(u_hbm, v_hbm, zs_hbm, zd_hbm, src_hbm, dst_hbm, wb_hbm,
               r_hbm, t_hbm, in_hbm,
               sidx, didx, p, zs, zd, wb, rbuf, tbuf, ibuf, semu, semv, semz):
    c = lax.axis_index("c")
    s = lax.axis_index("s")
    pltpu.sync_copy(wb_hbm, wb)
    base0 = (c * NS + s) * (EP // (NC * NS * CH))
    lanes = lax.iota(_i32, 16)

    def col(j):
        return jnp.full((16,), j, _i32)

    def chunk(k, carry):
        r0 = base0 + k * SUB
        b = r0 * CH
        pltpu.sync_copy(src_hbm.at[pl.ds(r0, SUB)], sidx)
        pltpu.sync_copy(dst_hbm.at[pl.ds(r0, SUB)], didx)
        du = [pltpu.async_copy(u_hbm.at[sidx.at[i]],
                               p.at[pl.ds(i * CH, CH)], semu)
              for i in range(SUB)]
        dz = [pltpu.async_copy(zs_hbm.at[sidx.at[i]],
                               zs.at[pl.ds(i * CH, CH)], semz)
              for i in range(SUB)]
        dz += [pltpu.async_copy(zd_hbm.at[didx.at[i]],
                                zd.at[pl.ds(i * CH, CH)], semz)
               for i in range(SUB)]
        for d in du:
            d.wait()
        dv = [pltpu.async_copy(v_hbm.at[didx.at[i]],
                               p.at[pl.ds(i * CH, CH)], semv, add=True)
              for i in range(SUB)]
        for d in dv:
            d.wait()
        for d in dz:
            d.wait()

        def group(g, carry2):
            row = g * 16 + lanes
            accr = jnp.zeros((16,), _f32)
            acct = jnp.zeros((16,), _f32)
            acci = jnp.zeros((16,), _f32)
            for j in range(64):
                x = plsc.load_gather(p, [row, col(j)])
                accr = accr + jnp.maximum(x, 0.2 * x) * wb[j]
            for j in range(64):
                x = plsc.load_gather(p, [row, col(64 + j)])
                acct = acct + jnp.maximum(x, 0.2 * x) * wb[64 + j]
            for j in range(17):
                a = plsc.load_gather(zs, [row, col(j)])
                bb = plsc.load_gather(zd, [row, col(j)])
                acci = acci + a * bb
            rbuf[pl.ds(g * 16, 16)] = accr
            tbuf[pl.ds(g * 16, 16)] = acct
            ibuf[pl.ds(g * 16, 16)] = acci
            return carry2

        lax.fori_loop(0, SUB * CH // 16, group, 0)
        pltpu.sync_copy(rbuf, r_hbm.at[pl.ds(b, SUB * CH)])
        pltpu.sync_copy(tbuf, t_hbm.at[pl.ds(b, SUB * CH)])
        pltpu.sync_copy(ibuf, in_hbm.at[pl.ds(b, SUB * CH)])
        return carry

    lax.fori_loop(0, EP // (NC * NS * CH * SUB), chunk, 0)


def _edge_call(U, V, Zs, Zd, src2d, dst2d, wb):
    f = pl.kernel(
        _edge_body,
        out_type=[jax.ShapeDtypeStruct((EP,), _f32)] * 3,
        mesh=_mesh(),
        scratch_types=[
            pltpu.VMEM((SUB, CH), _i32),
            pltpu.VMEM((SUB, CH), _i32),
            pltpu.VMEM((SUB * CH, 128), _f32),
            pltpu.VMEM((SUB * CH, 32), _f32),
            pltpu.VMEM((SUB * CH, 32), _f32),
            pltpu.VMEM((128, 16), _f32),
            pltpu.VMEM((SUB * CH,), _f32),
            pltpu.VMEM((SUB * CH,), _f32),
            pltpu.VMEM((SUB * CH,), _f32),
            pltpu.SemaphoreType.DMA,
            pltpu.SemaphoreType.DMA,
            pltpu.SemaphoreType.DMA,
        ],
        compiler_params=pltpu.CompilerParams(
            use_tc_tiling_on_sc=False, needs_layout_passes=False),
        name="sc_edge",
    )
    return f(U, V, Zs, Zd, src2d, dst2d, wb)


def _prep_body(z_ref, d0_ref, d1_ref, w0_ref, yp_ref, dinv_ref, zs_ref, zd_ref):
    z = z_ref[...]
    deg = d0_ref[0][:, :1] + d1_ref[0][:, :1] + 1.0
    dinv = lax.rsqrt(deg)
    dinv_ref[...] = dinv
    x0 = z[:, :1]
    alpha = jnp.maximum(x0 / RADIUS, 1.0 + 1e-7)
    acosh = jnp.log(alpha + jnp.sqrt(alpha * alpha - 1.0))
    coef = acosh / jnp.sqrt(alpha * alpha - 1.0)
    zmu = coef * jnp.concatenate([x0 - alpha * RADIUS, z[:, 1:]], axis=1)
    y = (zmu @ w0_ref[...]) * dinv
    yp_ref[0] = y[:, :32]
    yp_ref[1] = y[:, 32:]
    pad = jnp.zeros((z.shape[0], 15), _f32)
    zs_ref[...] = jnp.concatenate([-x0, z[:, 1:], pad], axis=1)
    zd_ref[...] = jnp.concatenate([x0, z[:, 1:], pad], axis=1)


def _prep_call(z, deg2, W0):
    return pl.pallas_call(
        _prep_body,
        grid=(GRID,),
        in_specs=[
            pl.BlockSpec((B, 17), lambda i: (i, 0)),
            pl.BlockSpec((1, B, 16), lambda i: (0, i, 0)),
            pl.BlockSpec((1, B, 16), lambda i: (1, i, 0)),
            pl.BlockSpec((17, 64), lambda i: (0, 0)),
        ],
        out_specs=[
            pl.BlockSpec((2, B, 32), lambda i: (0, i, 0)),
            pl.BlockSpec((B, 1), lambda i: (i, 0)),
            pl.BlockSpec((B, 32), lambda i: (i, 0)),
            pl.BlockSpec((B, 32), lambda i: (i, 0)),
        ],
        out_shape=[
            jax.ShapeDtypeStruct((2, N, 32), _f32),
            jax.ShapeDtypeStruct((N, 1), _f32),
            jax.ShapeDtypeStruct((N, 32), _f32),
            jax.ShapeDtypeStruct((N, 32), _f32),
        ],
        name="tc_prep",
    )(z, deg2, deg2, W0)


def _layer_body(acc_ref, yp_ref, dinv_ref, b_ref, w_ref, out_ref, *, relu, wo2):
    y = jnp.concatenate([acc_ref[0] + yp_ref[0], acc_ref[1] + yp_ref[1]], axis=1)
    h = dinv_ref[...] * y + b_ref[...]
    if relu:
        h = jnp.maximum(h, 0.0)
    yn = (h @ w_ref[...]) * dinv_ref[...]
    out_ref[0] = yn[:, :wo2]
    out_ref[1] = yn[:, wo2:]


def _layer_call(acc, yp, dinv, bias, W, relu):
    wi2 = yp.shape[2]
    wo2 = W.shape[1] // 2
    body = functools.partial(_layer_body, relu=relu, wo2=wo2)
    return pl.pallas_call(
        body,
        grid=(GRID,),
        in_specs=[
            pl.BlockSpec((2, B, wi2), lambda i: (0, i, 0)),
            pl.BlockSpec((2, B, wi2), lambda i: (0, i, 0)),
            pl.BlockSpec((B, 1), lambda i: (i, 0)),
            pl.BlockSpec((2 * wi2,), lambda i: (0,)),
            pl.BlockSpec(W.shape, lambda i: (0, 0)),
        ],
        out_specs=pl.BlockSpec((2, B, wo2), lambda i: (0, i, 0)),
        out_shape=jax.ShapeDtypeStruct((2, N, wo2), _f32),
        name=f"tc_layer{W.shape[1]}",
    )(acc, yp, dinv, bias, W)


def _final_node_body(acc_ref, yp_ref, dinv_ref, b_ref, wu_ref, wv_ref, bv_ref,
                     u_ref, v_ref):
    y = jnp.concatenate([acc_ref[0] + yp_ref[0], acc_ref[1] + yp_ref[1]], axis=1)
    g = dinv_ref[...] * y + b_ref[...]
    u_ref[...] = g @ wu_ref[...]
    v_ref[...] = g @ wv_ref[...] + bv_ref[...]


def _final_node_call(acc, yp, dinv, b2, WU, WV, bV):
    return pl.pallas_call(
        _final_node_body,
        grid=(GRID,),
        in_specs=[
            pl.BlockSpec((2, B, 16), lambda i: (0, i, 0)),
            pl.BlockSpec((2, B, 16), lambda i: (0, i, 0)),
            pl.BlockSpec((B, 1), lambda i: (i, 0)),
            pl.BlockSpec((32,), lambda i: (0,)),
            pl.BlockSpec((32, 128), lambda i: (0, 0)),
            pl.BlockSpec((32, 128), lambda i: (0, 0)),
            pl.BlockSpec((128,), lambda i: (0,)),
        ],
        out_specs=[
            pl.BlockSpec((B, 128), lambda i: (i, 0)),
            pl.BlockSpec((B, 128), lambda i: (i, 0)),
        ],
        out_shape=[
            jax.ShapeDtypeStruct((N, 128), _f32),
            jax.ShapeDtypeStruct((N, 128), _f32),
        ],
        name="tc_final_node",
    )(acc, yp, dinv, b2, WU, WV, bV)


def _combine_body(in_ref, r_ref, t_ref, br_ref, bt_ref, out_ref):
    arg = jnp.maximum(-in_ref[...] / (RADIUS * RADIUS), 1.0 + 1e-7)
    dist = -RADIUS * jnp.log(arg + jnp.sqrt(arg * arg - 1.0))
    r = r_ref[...] + br_ref[0]
    t = t_ref[...] + bt_ref[0]
    x = (dist - r) / t
    out_ref[...] = 1.0 / (1.0 + jnp.exp(-x))


def _combine_call(inner, rpre, tpre, br2, bt2):
    rows = EP // 128
    blk = 64
    return pl.pallas_call(
        _combine_body,
        grid=(rows // blk,),
        in_specs=[
            pl.BlockSpec((blk, 128), lambda i: (i, 0)),
            pl.BlockSpec((blk, 128), lambda i: (i, 0)),
            pl.BlockSpec((blk, 128), lambda i: (i, 0)),
            pl.BlockSpec(memory_space=pltpu.SMEM),
            pl.BlockSpec(memory_space=pltpu.SMEM),
        ],
        out_specs=pl.BlockSpec((blk, 128), lambda i: (i, 0)),
        out_shape=jax.ShapeDtypeStruct((rows, 128), _f32),
        name="tc_combine",
    )(inner.reshape(rows, 128), rpre.reshape(rows, 128),
      tpre.reshape(rows, 128), br2, bt2)


def kernel(z, edge_index, W0, b0, W1, b1, W2, b2,
           Wr1, br1, Wr2, br2, Wt1, bt1, Wt2, bt2):
    src = edge_index[0]
    dst = edge_index[1]
    padn = EP - E
    zero_pad = jnp.zeros((padn,), _i32)
    src_p = jnp.concatenate([src, zero_pad])
    dst_e = jnp.concatenate([dst, zero_pad])
    dst_seg = jnp.concatenate([dst, jnp.full((padn,), N, _i32)])

    src2d = src_p.reshape(EP // CH, CH)
    dst2d_seg = dst_seg.reshape(EP // CH, CH)
    dst2d_e = dst_e.reshape(EP // CH, CH)

    deg2 = _deg_call(dst_seg).reshape(2, NPAD, 16)
    yp0, dinv, Zs, Zd = _prep_call(z, deg2, W0)

    acc0 = _seg_call(yp0, src2d, dst2d_seg, 32).reshape(2, NPAD, 32)
    yp1 = _layer_call(acc0, yp0, dinv, b0, W1, relu=True)
    acc1 = _seg_call(yp1, src2d, dst2d_seg, 32).reshape(2, NPAD, 32)
    yp2 = _layer_call(acc1, yp1, dinv, b1, W2, relu=True)
    acc2 = _seg_call(yp2, src2d, dst2d_seg, 16).reshape(2, NPAD, 16)

    WU = jnp.concatenate([Wr1[:32], Wt1[:32]], axis=1)
    WV = jnp.concatenate([Wr1[32:], Wt1[32:]], axis=1)
    bV = jnp.concatenate([br1, bt1])
    U, V = _final_node_call(acc2, yp2, dinv, b2, WU, WV, bV)

    wb = jnp.concatenate([jnp.broadcast_to(Wr2, (64, 16)),
                          jnp.broadcast_to(Wt2, (64, 16))], axis=0)
    rpre, tpre, inner = _edge_call(U, V, Zs, Zd, src2d, dst2d_e, wb)
    probs = _combine_call(inner, rpre, tpre, br2, bt2)
    return probs.reshape(EP)[:E]


# speedup vs baseline: 5.5554x; 1.0458x over previous
import functools

import jax
import jax.numpy as jnp
from jax import lax
from jax.experimental import pallas as pl
from jax.experimental.pallas import tpu as pltpu
from jax.experimental.pallas import tpu_sc as plsc

N = 50000
E = 800000
EP = 819200
CH = 128
NPAD = 50176
STRIPE = 3136
ZCH = 448
NC = 2
NS = 16
SUB = 4
B = 1000
GRID = N // B
RADIUS = 1.0

_f32 = jnp.float32
_i32 = jnp.int32


def _mesh():
    return plsc.VectorSubcoreMesh(core_axis_name="c", subcore_axis_name="s")


def _deg_body(dst_hbm, zeros_hbm, ones_hbm, out_hbm, ones_v, idx_v, acc):
    c = lax.axis_index("c")
    s = lax.axis_index("s")
    for k in range(STRIPE // ZCH):
        pltpu.sync_copy(zeros_hbm, acc.at[pl.ds(s * STRIPE + k * ZCH, ZCH)])
    plsc.subcore_barrier()
    pltpu.sync_copy(ones_hbm, ones_v)
    base0 = (c * NS + s) * (EP // (NC * NS))

    def chunk(k, carry):
        b = base0 + k * CH
        pltpu.sync_copy(dst_hbm.at[pl.ds(b, CH)], idx_v)
        pltpu.sync_copy(ones_v, acc.at[idx_v], add=True)
        return carry

    lax.fori_loop(0, EP // (NC * NS * CH), chunk, 0)
    plsc.subcore_barrier()
    pltpu.sync_copy(acc.at[pl.ds(s * STRIPE, STRIPE)],
                    out_hbm.at[pl.ds(c * NPAD + s * STRIPE, STRIPE)])


def _deg_call(dst_seg):
    f = pl.kernel(
        _deg_body,
        out_type=jax.ShapeDtypeStruct((2 * NPAD, 16), _f32),
        mesh=_mesh(),
        scratch_types=[
            pltpu.VMEM((CH, 16), _f32),
            pltpu.VMEM((CH,), _i32),
            pltpu.VMEM_SHARED((NPAD, 16), _f32),
        ],
        compiler_params=pltpu.CompilerParams(use_tc_tiling_on_sc=False),
        name="sc_degree",
    )
    return f(dst_seg, jnp.zeros((ZCH, 16), _f32), jnp.ones((CH, 16), _f32))


def _seg_body(taba_hbm, tabb_hbm, src_hbm, dst_hbm, zeros_hbm, out_hbm,
              sidx, didx, rows, acc, sem):
    c = lax.axis_index("c")
    s = lax.axis_index("s")
    for k in range(STRIPE // ZCH):
        pltpu.sync_copy(zeros_hbm, acc.at[pl.ds(s * STRIPE + k * ZCH, ZCH)])
    plsc.subcore_barrier()
    base0 = s * (EP // (NS * CH))

    def chunk(k, carry):
        r0 = base0 + k * SUB
        pltpu.sync_copy(src_hbm.at[pl.ds(r0, SUB)], sidx)
        pltpu.sync_copy(dst_hbm.at[pl.ds(r0, SUB)], didx)

        @pl.when(c == 0)
        def _():
            ds = [pltpu.async_copy(taba_hbm.at[sidx.at[i]],
                                   rows.at[pl.ds(i * CH, CH)], sem)
                  for i in range(SUB)]
            for d in ds:
                d.wait()

        @pl.when(c == 1)
        def _():
            ds = [pltpu.async_copy(tabb_hbm.at[sidx.at[i]],
                                   rows.at[pl.ds(i * CH, CH)], sem)
                  for i in range(SUB)]
            for d in ds:
                d.wait()

        for i in range(SUB):
            pltpu.sync_copy(rows.at[pl.ds(i * CH, CH)], acc.at[didx.at[i]],
                            add=True)
        return carry

    lax.fori_loop(0, EP // (NS * CH * SUB), chunk, 0)
    plsc.subcore_barrier()
    pltpu.sync_copy(acc.at[pl.ds(s * STRIPE, STRIPE)],
                    out_hbm.at[pl.ds(c * NPAD + s * STRIPE, STRIPE)])


def _seg_call(tab2, src2d, dst2d, w2):
    f = pl.kernel(
        _seg_body,
        out_type=jax.ShapeDtypeStruct((2 * NPAD, w2), _f32),
        mesh=_mesh(),
        scratch_types=[
            pltpu.VMEM((SUB, CH), _i32),
            pltpu.VMEM((SUB, CH), _i32),
            pltpu.VMEM((SUB * CH, w2), _f32),
            pltpu.VMEM_SHARED((NPAD, w2), _f32),
            pltpu.SemaphoreType.DMA,
        ],
        compiler_params=pltpu.CompilerParams(use_tc_tiling_on_sc=False),
        name=f"sc_segsum{w2}",
    )
    return f(tab2[0], tab2[1], src2d, dst2d, jnp.zeros((ZCH, w2), _f32))


def _edge_body(u_hbm, v_hbm, zs_hbm, zd_hbm, src_hbm, dst_hbm, wb_hbm,
               r_hbm, t_hbm, in_hbm,
               sidx, didx, pu, pv, zs, zd, wb, rbuf, tbuf, ibuf, sem):
    c = lax.axis_index("c")
    s = lax.axis_index("s")
    pltpu.sync_copy(wb_hbm, wb)
    base0 = (c * NS + s) * (EP // (NC * NS * CH))
    lanes = lax.iota(_i32, 16)

    def col(j):
        return jnp.full((16,), j, _i32)

    def unpk(x):
        return plsc.unpack(plsc.bitcast(x, jnp.bfloat16),
                           format=plsc.PackFormat.INTERLEAVED)

    def chunk(k, carry):
        r0 = base0 + k * SUB
        b = r0 * CH
        pltpu.sync_copy(src_hbm.at[pl.ds(r0, SUB)], sidx)
        pltpu.sync_copy(dst_hbm.at[pl.ds(r0, SUB)], didx)
        dd = [pltpu.async_copy(u_hbm.at[sidx.at[i]],
                               pu.at[pl.ds(i * CH, CH)], sem)
              for i in range(SUB)]
        dd += [pltpu.async_copy(v_hbm.at[didx.at[i]],
                                pv.at[pl.ds(i * CH, CH)], sem)
               for i in range(SUB)]
        dd += [pltpu.async_copy(zs_hbm.at[sidx.at[i]],
                                zs.at[pl.ds(i * CH, CH)], sem)
               for i in range(SUB)]
        dd += [pltpu.async_copy(zd_hbm.at[didx.at[i]],
                                zd.at[pl.ds(i * CH, CH)], sem)
               for i in range(SUB)]
        for d in dd:
            d.wait()

        def group(g, carry2):
            row = g * 16 + lanes
            accr = jnp.zeros((16,), _f32)
            acct = jnp.zeros((16,), _f32)
            acci = jnp.zeros((16,), _f32)
            for w in range(32):
                xu = plsc.load_gather(pu, [row, col(w)])
                xv = plsc.load_gather(pv, [row, col(w)])
                ua, ub = unpk(xu)
                va, vb = unpk(xv)
                a = ua + va
                bb = ub + vb
                accr = accr + jnp.maximum(a, 0.2 * a) * wb[2 * w]
                accr = accr + jnp.maximum(bb, 0.2 * bb) * wb[2 * w + 1]
            for w in range(32):
                xu = plsc.load_gather(pu, [row, col(32 + w)])
                xv = plsc.load_gather(pv, [row, col(32 + w)])
                ua, ub = unpk(xu)
                va, vb = unpk(xv)
                a = ua + va
                bb = ub + vb
                acct = acct + jnp.maximum(a, 0.2 * a) * wb[64 + 2 * w]
                acct = acct + jnp.maximum(bb, 0.2 * bb) * wb[64 + 2 * w + 1]
            for j in range(17):
                a = plsc.load_gather(zs, [row, col(j)])
                bb = plsc.load_gather(zd, [row, col(j)])
                acci = acci + a * bb
            rbuf[pl.ds(g * 16, 16)] = accr
            tbuf[pl.ds(g * 16, 16)] = acct
            ibuf[pl.ds(g * 16, 16)] = acci
            return carry2

        lax.fori_loop(0, SUB * CH // 16, group, 0)
        pltpu.sync_copy(rbuf, r_hbm.at[pl.ds(b, SUB * CH)])
        pltpu.sync_copy(tbuf, t_hbm.at[pl.ds(b, SUB * CH)])
        pltpu.sync_copy(ibuf, in_hbm.at[pl.ds(b, SUB * CH)])
        return carry

    lax.fori_loop(0, EP // (NC * NS * CH * SUB), chunk, 0)


def _edge_call(U32, V32, Zs, Zd, src2d, dst2d, wb):
    f = pl.kernel(
        _edge_body,
        out_type=[jax.ShapeDtypeStruct((EP,), _f32)] * 3,
        mesh=_mesh(),
        scratch_types=[
            pltpu.VMEM((SUB, CH), _i32),
            pltpu.VMEM((SUB, CH), _i32),
            pltpu.VMEM((SUB * CH, 64), _i32),
            pltpu.VMEM((SUB * CH, 64), _i32),
            pltpu.VMEM((SUB * CH, 32), _f32),
            pltpu.VMEM((SUB * CH, 32), _f32),
            pltpu.VMEM((128, 16), _f32),
            pltpu.VMEM((SUB * CH,), _f32),
            pltpu.VMEM((SUB * CH,), _f32),
            pltpu.VMEM((SUB * CH,), _f32),
            pltpu.SemaphoreType.DMA,
        ],
        compiler_params=pltpu.CompilerParams(
            use_tc_tiling_on_sc=False, needs_layout_passes=False),
        name="sc_edge",
    )
    return f(U32, V32, Zs, Zd, src2d, dst2d, wb)


def _prep_body(z_ref, d0_ref, d1_ref, w0_ref, yp_ref, dinv_ref, zs_ref, zd_ref):
    z = z_ref[...]
    deg = d0_ref[0][:, :1] + d1_ref[0][:, :1] + 1.0
    dinv = lax.rsqrt(deg)
    dinv_ref[...] = dinv
    x0 = z[:, :1]
    alpha = jnp.maximum(x0 / RADIUS, 1.0 + 1e-7)
    acosh = jnp.log(alpha + jnp.sqrt(alpha * alpha - 1.0))
    coef = acosh / jnp.sqrt(alpha * alpha - 1.0)
    zmu = coef * jnp.concatenate([x0 - alpha * RADIUS, z[:, 1:]], axis=1)
    y = (zmu @ w0_ref[...]) * dinv
    yp_ref[0] = y[:, :32]
    yp_ref[1] = y[:, 32:]
    pad = jnp.zeros((z.shape[0], 15), _f32)
    zs_ref[...] = jnp.concatenate([-x0, z[:, 1:], pad], axis=1)
    zd_ref[...] = jnp.concatenate([x0, z[:, 1:], pad], axis=1)


def _prep_call(z, deg2, W0):
    return pl.pallas_call(
        _prep_body,
        grid=(GRID,),
        in_specs=[
            pl.BlockSpec((B, 17), lambda i: (i, 0)),
            pl.BlockSpec((1, B, 16), lambda i: (0, i, 0)),
            pl.BlockSpec((1, B, 16), lambda i: (1, i, 0)),
            pl.BlockSpec((17, 64), lambda i: (0, 0)),
        ],
        out_specs=[
            pl.BlockSpec((2, B, 32), lambda i: (0, i, 0)),
            pl.BlockSpec((B, 1), lambda i: (i, 0)),
            pl.BlockSpec((B, 32), lambda i: (i, 0)),
            pl.BlockSpec((B, 32), lambda i: (i, 0)),
        ],
        out_shape=[
            jax.ShapeDtypeStruct((2, N, 32), _f32),
            jax.ShapeDtypeStruct((N, 1), _f32),
            jax.ShapeDtypeStruct((N, 32), _f32),
            jax.ShapeDtypeStruct((N, 32), _f32),
        ],
        name="tc_prep",
    )(z, deg2, deg2, W0)


def _layer_body(acc_ref, yp_ref, dinv_ref, b_ref, w_ref, out_ref, *, relu, wo2):
    y = jnp.concatenate([acc_ref[0] + yp_ref[0], acc_ref[1] + yp_ref[1]], axis=1)
    h = dinv_ref[...] * y + b_ref[...]
    if relu:
        h = jnp.maximum(h, 0.0)
    yn = (h @ w_ref[...]) * dinv_ref[...]
    out_ref[0] = yn[:, :wo2]
    out_ref[1] = yn[:, wo2:]


def _layer_call(acc, yp, dinv, bias, W, relu):
    wi2 = yp.shape[2]
    wo2 = W.shape[1] // 2
    body = functools.partial(_layer_body, relu=relu, wo2=wo2)
    return pl.pallas_call(
        body,
        grid=(GRID,),
        in_specs=[
            pl.BlockSpec((2, B, wi2), lambda i: (0, i, 0)),
            pl.BlockSpec((2, B, wi2), lambda i: (0, i, 0)),
            pl.BlockSpec((B, 1), lambda i: (i, 0)),
            pl.BlockSpec((2 * wi2,), lambda i: (0,)),
            pl.BlockSpec(W.shape, lambda i: (0, 0)),
        ],
        out_specs=pl.BlockSpec((2, B, wo2), lambda i: (0, i, 0)),
        out_shape=jax.ShapeDtypeStruct((2, N, wo2), _f32),
        name=f"tc_layer{W.shape[1]}",
    )(acc, yp, dinv, bias, W)


def _final_node_body(acc_ref, yp_ref, dinv_ref, b_ref, wu_ref, wv_ref, bv_ref,
                     u_ref, v_ref):
    y = jnp.concatenate([acc_ref[0] + yp_ref[0], acc_ref[1] + yp_ref[1]], axis=1)
    g = dinv_ref[...] * y + b_ref[...]
    u_ref[...] = (g @ wu_ref[...]).astype(jnp.bfloat16)
    v_ref[...] = (g @ wv_ref[...] + bv_ref[...]).astype(jnp.bfloat16)


def _final_node_call(acc, yp, dinv, b2, WU, WV, bV):
    return pl.pallas_call(
        _final_node_body,
        grid=(GRID,),
        in_specs=[
            pl.BlockSpec((2, B, 16), lambda i: (0, i, 0)),
            pl.BlockSpec((2, B, 16), lambda i: (0, i, 0)),
            pl.BlockSpec((B, 1), lambda i: (i, 0)),
            pl.BlockSpec((32,), lambda i: (0,)),
            pl.BlockSpec((32, 128), lambda i: (0, 0)),
            pl.BlockSpec((32, 128), lambda i: (0, 0)),
            pl.BlockSpec((128,), lambda i: (0,)),
        ],
        out_specs=[
            pl.BlockSpec((B, 128), lambda i: (i, 0)),
            pl.BlockSpec((B, 128), lambda i: (i, 0)),
        ],
        out_shape=[
            jax.ShapeDtypeStruct((N, 128), jnp.bfloat16),
            jax.ShapeDtypeStruct((N, 128), jnp.bfloat16),
        ],
        name="tc_final_node",
    )(acc, yp, dinv, b2, WU, WV, bV)


def _combine_body(in_ref, r_ref, t_ref, br_ref, bt_ref, out_ref):
    arg = jnp.maximum(-in_ref[...] / (RADIUS * RADIUS), 1.0 + 1e-7)
    dist = -RADIUS * jnp.log(arg + jnp.sqrt(arg * arg - 1.0))
    r = r_ref[...] + br_ref[0]
    t = t_ref[...] + bt_ref[0]
    x = (dist - r) / t
    out_ref[...] = 1.0 / (1.0 + jnp.exp(-x))


def _combine_call(inner, rpre, tpre, br2, bt2):
    rows = EP // 128
    blk = 64
    return pl.pallas_call(
        _combine_body,
        grid=(rows // blk,),
        in_specs=[
            pl.BlockSpec((blk, 128), lambda i: (i, 0)),
            pl.BlockSpec((blk, 128), lambda i: (i, 0)),
            pl.BlockSpec((blk, 128), lambda i: (i, 0)),
            pl.BlockSpec(memory_space=pltpu.SMEM),
            pl.BlockSpec(memory_space=pltpu.SMEM),
        ],
        out_specs=pl.BlockSpec((blk, 128), lambda i: (i, 0)),
        out_shape=jax.ShapeDtypeStruct((rows, 128), _f32),
        name="tc_combine",
    )(inner.reshape(rows, 128), rpre.reshape(rows, 128),
      tpre.reshape(rows, 128), br2, bt2)


def kernel(z, edge_index, W0, b0, W1, b1, W2, b2,
           Wr1, br1, Wr2, br2, Wt1, bt1, Wt2, bt2):
    src = edge_index[0]
    dst = edge_index[1]
    padn = EP - E
    zero_pad = jnp.zeros((padn,), _i32)
    src_p = jnp.concatenate([src, zero_pad])
    dst_e = jnp.concatenate([dst, zero_pad])
    dst_seg = jnp.concatenate([dst, jnp.full((padn,), N, _i32)])

    src2d = src_p.reshape(EP // CH, CH)
    dst2d_seg = dst_seg.reshape(EP // CH, CH)
    dst2d_e = dst_e.reshape(EP // CH, CH)

    deg2 = _deg_call(dst_seg).reshape(2, NPAD, 16)
    yp0, dinv, Zs, Zd = _prep_call(z, deg2, W0)

    acc0 = _seg_call(yp0, src2d, dst2d_seg, 32).reshape(2, NPAD, 32)
    yp1 = _layer_call(acc0, yp0, dinv, b0, W1, relu=True)
    acc1 = _seg_call(yp1, src2d, dst2d_seg, 32).reshape(2, NPAD, 32)
    yp2 = _layer_call(acc1, yp1, dinv, b1, W2, relu=True)
    acc2 = _seg_call(yp2, src2d, dst2d_seg, 16).reshape(2, NPAD, 16)

    WU = jnp.concatenate([Wr1[:32], Wt1[:32]], axis=1)
    WV = jnp.concatenate([Wr1[32:], Wt1[32:]], axis=1)
    bV = jnp.concatenate([br1, bt1])
    U, V = _final_node_call(acc2, yp2, dinv, b2, WU, WV, bV)
    U32 = lax.bitcast_convert_type(U.reshape(N, 64, 2), _i32)
    V32 = lax.bitcast_convert_type(V.reshape(N, 64, 2), _i32)

    wb = jnp.concatenate([jnp.broadcast_to(Wr2, (64, 16)),
                          jnp.broadcast_to(Wt2, (64, 16))], axis=0)
    rpre, tpre, inner = _edge_call(U32, V32, Zs, Zd, src2d, dst2d_e, wb)
    probs = _combine_call(inner, rpre, tpre, br2, bt2)
    return probs.reshape(EP)[:E]


# speedup vs baseline: 6.2394x; 1.1231x over previous
import functools

import jax
import jax.numpy as jnp
from jax import lax
from jax.experimental import pallas as pl
from jax.experimental.pallas import tpu as pltpu
from jax.experimental.pallas import tpu_sc as plsc

N = 50000
E = 800000
EP = 819200
CH = 128
NPAD = 50176
STRIPE = 3136
ZCH = 448
NC = 2
NS = 16
SUB = 4
B = 1000
GRID = N // B
RADIUS = 1.0

_f32 = jnp.float32
_i32 = jnp.int32


def _mesh():
    return plsc.VectorSubcoreMesh(core_axis_name="c", subcore_axis_name="s")


def _deg_body(dst_hbm, zeros_hbm, ones_hbm, out_hbm, ones_v, idx_v, acc):
    c = lax.axis_index("c")
    s = lax.axis_index("s")
    for k in range(STRIPE // ZCH):
        pltpu.sync_copy(zeros_hbm, acc.at[pl.ds(s * STRIPE + k * ZCH, ZCH)])
    plsc.subcore_barrier()
    pltpu.sync_copy(ones_hbm, ones_v)
    base0 = (c * NS + s) * (EP // (NC * NS))

    def chunk(k, carry):
        b = base0 + k * CH
        pltpu.sync_copy(dst_hbm.at[pl.ds(b, CH)], idx_v)
        pltpu.sync_copy(ones_v, acc.at[idx_v], add=True)
        return carry

    lax.fori_loop(0, EP // (NC * NS * CH), chunk, 0)
    plsc.subcore_barrier()
    pltpu.sync_copy(acc.at[pl.ds(s * STRIPE, STRIPE)],
                    out_hbm.at[pl.ds(c * NPAD + s * STRIPE, STRIPE)])


def _deg_call(dst_seg):
    f = pl.kernel(
        _deg_body,
        out_type=jax.ShapeDtypeStruct((2 * NPAD, 16), _f32),
        mesh=_mesh(),
        scratch_types=[
            pltpu.VMEM((CH, 16), _f32),
            pltpu.VMEM((CH,), _i32),
            pltpu.VMEM_SHARED((NPAD, 16), _f32),
        ],
        compiler_params=pltpu.CompilerParams(use_tc_tiling_on_sc=False),
        name="sc_degree",
    )
    return f(dst_seg, jnp.zeros((ZCH, 16), _f32), jnp.ones((CH, 16), _f32))


def _seg_body(taba_hbm, tabb_hbm, src_hbm, dst_hbm, zeros_hbm, out_hbm,
              sidx, didx, rows, acc, sem):
    c = lax.axis_index("c")
    s = lax.axis_index("s")
    for k in range(STRIPE // ZCH):
        pltpu.sync_copy(zeros_hbm, acc.at[pl.ds(s * STRIPE + k * ZCH, ZCH)])
    plsc.subcore_barrier()
    base0 = s * (EP // (NS * CH))

    def chunk(k, carry):
        r0 = base0 + k * SUB
        pltpu.sync_copy(src_hbm.at[pl.ds(r0, SUB)], sidx)
        pltpu.sync_copy(dst_hbm.at[pl.ds(r0, SUB)], didx)

        @pl.when(c == 0)
        def _():
            ds = [pltpu.async_copy(taba_hbm.at[sidx.at[i]],
                                   rows.at[pl.ds(i * CH, CH)], sem)
                  for i in range(SUB)]
            for d in ds:
                d.wait()

        @pl.when(c == 1)
        def _():
            ds = [pltpu.async_copy(tabb_hbm.at[sidx.at[i]],
                                   rows.at[pl.ds(i * CH, CH)], sem)
                  for i in range(SUB)]
            for d in ds:
                d.wait()

        for i in range(SUB):
            pltpu.sync_copy(rows.at[pl.ds(i * CH, CH)], acc.at[didx.at[i]],
                            add=True)
        return carry

    lax.fori_loop(0, EP // (NS * CH * SUB), chunk, 0)
    plsc.subcore_barrier()
    pltpu.sync_copy(acc.at[pl.ds(s * STRIPE, STRIPE)],
                    out_hbm.at[pl.ds(c * NPAD + s * STRIPE, STRIPE)])


def _seg_call(tab2, src2d, dst2d, w2):
    f = pl.kernel(
        _seg_body,
        out_type=jax.ShapeDtypeStruct((2 * NPAD, w2), _f32),
        mesh=_mesh(),
        scratch_types=[
            pltpu.VMEM((SUB, CH), _i32),
            pltpu.VMEM((SUB, CH), _i32),
            pltpu.VMEM((SUB * CH, w2), _f32),
            pltpu.VMEM_SHARED((NPAD, w2), _f32),
            pltpu.SemaphoreType.DMA,
        ],
        compiler_params=pltpu.CompilerParams(use_tc_tiling_on_sc=False),
        name=f"sc_segsum{w2}",
    )
    return f(tab2[0], tab2[1], src2d, dst2d, jnp.zeros((ZCH, w2), _f32))


ESUB = 2
ECH = ESUB * CH
ENCH = EP // (NC * NS * ECH)


def _edge_body(us_hbm, vd_hbm, src_hbm, dst_hbm, wb_hbm,
               r_hbm, t_hbm, in_hbm,
               sidx0, didx0, sidx1, didx1, pus0, pvd0, pus1, pvd1,
               wb, rbuf, tbuf, ibuf, semg0, semg1, semi0, semi1):
    c = lax.axis_index("c")
    s = lax.axis_index("s")
    pltpu.sync_copy(wb_hbm, wb)
    base0 = (c * NS + s) * (EP // (NC * NS * CH))
    lanes = lax.iota(_i32, 16)
    idxs = [(sidx0, didx0), (sidx1, didx1)]
    bufs = [(pus0, pvd0), (pus1, pvd1)]
    semg = [semg0, semg1]
    semi = [semi0, semi1]

    def col(j):
        return jnp.full((16,), j, _i32)

    def unpk(x):
        return plsc.unpack(plsc.bitcast(x, jnp.bfloat16),
                           format=plsc.PackFormat.INTERLEAVED)

    def issue_idx(k, P):
        r0 = base0 + k * ESUB
        si, di = idxs[P]
        pltpu.async_copy(src_hbm.at[pl.ds(r0, ESUB)], si, semi[P])
        pltpu.async_copy(dst_hbm.at[pl.ds(r0, ESUB)], di, semi[P])

    def wait_idx(P):
        si, di = idxs[P]
        pltpu.make_async_copy(src_hbm.at[pl.ds(0, ESUB)], si, semi[P]).wait()
        pltpu.make_async_copy(dst_hbm.at[pl.ds(0, ESUB)], di, semi[P]).wait()

    def issue_g(P):
        si, di = idxs[P]
        us, vd = bufs[P]
        for i in range(ESUB):
            pltpu.async_copy(us_hbm.at[si.at[i]],
                             us.at[pl.ds(i * CH, CH)], semg[P])
            pltpu.async_copy(vd_hbm.at[di.at[i]],
                             vd.at[pl.ds(i * CH, CH)], semg[P])

    def wait_g(P):
        si, di = idxs[P]
        us, vd = bufs[P]
        for i in range(ESUB):
            pltpu.make_async_copy(us_hbm.at[si.at[i]],
                                  us.at[pl.ds(i * CH, CH)], semg[P]).wait()
            pltpu.make_async_copy(vd_hbm.at[di.at[i]],
                                  vd.at[pl.ds(i * CH, CH)], semg[P]).wait()

    def compute(k, P):
        us, vd = bufs[P]
        b = (base0 + k * ESUB) * CH

        def group(g, carry2):
            row = g * 16 + lanes
            accr = jnp.zeros((16,), _f32)
            acct = jnp.zeros((16,), _f32)
            acci = jnp.zeros((16,), _f32)
            for w in range(64):
                xu = plsc.load_gather(us, [row, col(w)])
                xv = plsc.load_gather(vd, [row, col(w)])
                ua, ub = unpk(xu)
                va, vb = unpk(xv)
                a = ua + va
                bb = ub + vb
                acc = accr if w < 32 else acct
                acc = acc + jnp.maximum(a, 0.2 * a) * wb[2 * w]
                acc = acc + jnp.maximum(bb, 0.2 * bb) * wb[2 * w + 1]
                if w < 32:
                    accr = acc
                else:
                    acct = acc
            for j in range(17):
                a = plsc.bitcast(plsc.load_gather(us, [row, col(64 + j)]), _f32)
                bb = plsc.bitcast(plsc.load_gather(vd, [row, col(64 + j)]), _f32)
                acci = acci + a * bb
            rbuf[pl.ds(g * 16, 16)] = accr
            tbuf[pl.ds(g * 16, 16)] = acct
            ibuf[pl.ds(g * 16, 16)] = acci
            return carry2

        lax.fori_loop(0, ECH // 16, group, 0)
        pltpu.sync_copy(rbuf, r_hbm.at[pl.ds(b, ECH)])
        pltpu.sync_copy(tbuf, t_hbm.at[pl.ds(b, ECH)])
        pltpu.sync_copy(ibuf, in_hbm.at[pl.ds(b, ECH)])

    issue_idx(0, 0)
    wait_idx(0)
    issue_g(0)
    issue_idx(1, 1)

    def body(i, carry):
        k = 2 * i
        wait_idx(1)
        issue_g(1)
        wait_g(0)
        issue_idx(k + 2, 0)
        compute(k, 0)
        wait_idx(0)
        issue_g(0)
        wait_g(1)
        issue_idx(k + 3, 1)
        compute(k + 1, 1)
        return carry

    lax.fori_loop(0, ENCH // 2, body, 0)
    wait_g(0)
    wait_idx(1)


def _edge_call(US, VD, src2d, dst2d, wb):
    f = pl.kernel(
        _edge_body,
        out_type=[jax.ShapeDtypeStruct((EP,), _f32)] * 3,
        mesh=_mesh(),
        scratch_types=[
            pltpu.VMEM((ESUB, CH), _i32),
            pltpu.VMEM((ESUB, CH), _i32),
            pltpu.VMEM((ESUB, CH), _i32),
            pltpu.VMEM((ESUB, CH), _i32),
            pltpu.VMEM((ECH, 96), _i32),
            pltpu.VMEM((ECH, 96), _i32),
            pltpu.VMEM((ECH, 96), _i32),
            pltpu.VMEM((ECH, 96), _i32),
            pltpu.VMEM((128, 16), _f32),
            pltpu.VMEM((ECH,), _f32),
            pltpu.VMEM((ECH,), _f32),
            pltpu.VMEM((ECH,), _f32),
            pltpu.SemaphoreType.DMA,
            pltpu.SemaphoreType.DMA,
            pltpu.SemaphoreType.DMA,
            pltpu.SemaphoreType.DMA,
        ],
        compiler_params=pltpu.CompilerParams(
            use_tc_tiling_on_sc=False, needs_layout_passes=False),
        name="sc_edge",
    )
    return f(US, VD, src2d, dst2d, wb)


def _prep_body(z_ref, d0_ref, d1_ref, w0_ref, yp_ref, dinv_ref, zs_ref, zd_ref):
    z = z_ref[...]
    deg = d0_ref[0][:, :1] + d1_ref[0][:, :1] + 1.0
    dinv = lax.rsqrt(deg)
    dinv_ref[...] = dinv
    x0 = z[:, :1]
    alpha = jnp.maximum(x0 / RADIUS, 1.0 + 1e-7)
    acosh = jnp.log(alpha + jnp.sqrt(alpha * alpha - 1.0))
    coef = acosh / jnp.sqrt(alpha * alpha - 1.0)
    zmu = coef * jnp.concatenate([x0 - alpha * RADIUS, z[:, 1:]], axis=1)
    y = (zmu @ w0_ref[...]) * dinv
    yp_ref[0] = y[:, :32]
    yp_ref[1] = y[:, 32:]
    pad = jnp.zeros((z.shape[0], 15), _f32)
    zs_ref[...] = jnp.concatenate([-x0, z[:, 1:], pad], axis=1)
    zd_ref[...] = jnp.concatenate([x0, z[:, 1:], pad], axis=1)


def _prep_call(z, deg2, W0):
    return pl.pallas_call(
        _prep_body,
        grid=(GRID,),
        in_specs=[
            pl.BlockSpec((B, 17), lambda i: (i, 0)),
            pl.BlockSpec((1, B, 16), lambda i: (0, i, 0)),
            pl.BlockSpec((1, B, 16), lambda i: (1, i, 0)),
            pl.BlockSpec((17, 64), lambda i: (0, 0)),
        ],
        out_specs=[
            pl.BlockSpec((2, B, 32), lambda i: (0, i, 0)),
            pl.BlockSpec((B, 1), lambda i: (i, 0)),
            pl.BlockSpec((B, 32), lambda i: (i, 0)),
            pl.BlockSpec((B, 32), lambda i: (i, 0)),
        ],
        out_shape=[
            jax.ShapeDtypeStruct((2, N, 32), _f32),
            jax.ShapeDtypeStruct((N, 1), _f32),
            jax.ShapeDtypeStruct((N, 32), _f32),
            jax.ShapeDtypeStruct((N, 32), _f32),
        ],
        name="tc_prep",
    )(z, deg2, deg2, W0)


def _layer_body(acc_ref, yp_ref, dinv_ref, b_ref, w_ref, out_ref, *, relu, wo2):
    y = jnp.concatenate([acc_ref[0] + yp_ref[0], acc_ref[1] + yp_ref[1]], axis=1)
    h = dinv_ref[...] * y + b_ref[...]
    if relu:
        h = jnp.maximum(h, 0.0)
    yn = (h @ w_ref[...]) * dinv_ref[...]
    out_ref[0] = yn[:, :wo2]
    out_ref[1] = yn[:, wo2:]


def _layer_call(acc, yp, dinv, bias, W, relu):
    wi2 = yp.shape[2]
    wo2 = W.shape[1] // 2
    body = functools.partial(_layer_body, relu=relu, wo2=wo2)
    return pl.pallas_call(
        body,
        grid=(GRID,),
        in_specs=[
            pl.BlockSpec((2, B, wi2), lambda i: (0, i, 0)),
            pl.BlockSpec((2, B, wi2), lambda i: (0, i, 0)),
            pl.BlockSpec((B, 1), lambda i: (i, 0)),
            pl.BlockSpec((2 * wi2,), lambda i: (0,)),
            pl.BlockSpec(W.shape, lambda i: (0, 0)),
        ],
        out_specs=pl.BlockSpec((2, B, wo2), lambda i: (0, i, 0)),
        out_shape=jax.ShapeDtypeStruct((2, N, wo2), _f32),
        name=f"tc_layer{W.shape[1]}",
    )(acc, yp, dinv, bias, W)


def _final_node_body(acc_ref, yp_ref, dinv_ref, b_ref, wu_ref, wv_ref, bv_ref,
                     u_ref, v_ref):
    y = jnp.concatenate([acc_ref[0] + yp_ref[0], acc_ref[1] + yp_ref[1]], axis=1)
    g = dinv_ref[...] * y + b_ref[...]
    u_ref[...] = (g @ wu_ref[...]).astype(jnp.bfloat16)
    v_ref[...] = (g @ wv_ref[...] + bv_ref[...]).astype(jnp.bfloat16)


def _final_node_call(acc, yp, dinv, b2, WU, WV, bV):
    return pl.pallas_call(
        _final_node_body,
        grid=(GRID,),
        in_specs=[
            pl.BlockSpec((2, B, 16), lambda i: (0, i, 0)),
            pl.BlockSpec((2, B, 16), lambda i: (0, i, 0)),
            pl.BlockSpec((B, 1), lambda i: (i, 0)),
            pl.BlockSpec((32,), lambda i: (0,)),
            pl.BlockSpec((32, 128), lambda i: (0, 0)),
            pl.BlockSpec((32, 128), lambda i: (0, 0)),
            pl.BlockSpec((128,), lambda i: (0,)),
        ],
        out_specs=[
            pl.BlockSpec((B, 128), lambda i: (i, 0)),
            pl.BlockSpec((B, 128), lambda i: (i, 0)),
        ],
        out_shape=[
            jax.ShapeDtypeStruct((N, 128), jnp.bfloat16),
            jax.ShapeDtypeStruct((N, 128), jnp.bfloat16),
        ],
        name="tc_final_node",
    )(acc, yp, dinv, b2, WU, WV, bV)


def _combine_body(in_ref, r_ref, t_ref, br_ref, bt_ref, out_ref):
    arg = jnp.maximum(-in_ref[...] / (RADIUS * RADIUS), 1.0 + 1e-7)
    dist = -RADIUS * jnp.log(arg + jnp.sqrt(arg * arg - 1.0))
    r = r_ref[...] + br_ref[0]
    t = t_ref[...] + bt_ref[0]
    x = (dist - r) / t
    out_ref[...] = 1.0 / (1.0 + jnp.exp(-x))


def _combine_call(inner, rpre, tpre, br2, bt2):
    rows = EP // 128
    blk = 64
    return pl.pallas_call(
        _combine_body,
        grid=(rows // blk,),
        in_specs=[
            pl.BlockSpec((blk, 128), lambda i: (i, 0)),
            pl.BlockSpec((blk, 128), lambda i: (i, 0)),
            pl.BlockSpec((blk, 128), lambda i: (i, 0)),
            pl.BlockSpec(memory_space=pltpu.SMEM),
            pl.BlockSpec(memory_space=pltpu.SMEM),
        ],
        out_specs=pl.BlockSpec((blk, 128), lambda i: (i, 0)),
        out_shape=jax.ShapeDtypeStruct((rows, 128), _f32),
        name="tc_combine",
    )(inner.reshape(rows, 128), rpre.reshape(rows, 128),
      tpre.reshape(rows, 128), br2, bt2)


def kernel(z, edge_index, W0, b0, W1, b1, W2, b2,
           Wr1, br1, Wr2, br2, Wt1, bt1, Wt2, bt2):
    src = edge_index[0]
    dst = edge_index[1]
    padn = EP - E
    zero_pad = jnp.zeros((padn,), _i32)
    src_p = jnp.concatenate([src, zero_pad])
    dst_e = jnp.concatenate([dst, zero_pad])
    dst_seg = jnp.concatenate([dst, jnp.full((padn,), N, _i32)])

    extra = jnp.zeros((1024,), _i32)
    src2d = jnp.concatenate([src_p, extra]).reshape(-1, CH)
    dst2d_seg = jnp.concatenate([dst_seg, extra]).reshape(-1, CH)
    dst2d_e = jnp.concatenate([dst_e, extra]).reshape(-1, CH)

    deg2 = _deg_call(dst_seg).reshape(2, NPAD, 16)
    yp0, dinv, Zs, Zd = _prep_call(z, deg2, W0)

    acc0 = _seg_call(yp0, src2d, dst2d_seg, 32).reshape(2, NPAD, 32)
    yp1 = _layer_call(acc0, yp0, dinv, b0, W1, relu=True)
    acc1 = _seg_call(yp1, src2d, dst2d_seg, 32).reshape(2, NPAD, 32)
    yp2 = _layer_call(acc1, yp1, dinv, b1, W2, relu=True)
    acc2 = _seg_call(yp2, src2d, dst2d_seg, 16).reshape(2, NPAD, 16)

    WU = jnp.concatenate([Wr1[:32], Wt1[:32]], axis=1)
    WV = jnp.concatenate([Wr1[32:], Wt1[32:]], axis=1)
    bV = jnp.concatenate([br1, bt1])
    U, V = _final_node_call(acc2, yp2, dinv, b2, WU, WV, bV)
    U32 = lax.bitcast_convert_type(U.reshape(N, 64, 2), _i32)
    V32 = lax.bitcast_convert_type(V.reshape(N, 64, 2), _i32)
    US = jnp.concatenate([U32, lax.bitcast_convert_type(Zs, _i32)], axis=1)
    VD = jnp.concatenate([V32, lax.bitcast_convert_type(Zd, _i32)], axis=1)

    wb = jnp.concatenate([jnp.broadcast_to(Wr2, (64, 16)),
                          jnp.broadcast_to(Wt2, (64, 16))], axis=0)
    rpre, tpre, inner = _edge_call(US, VD, src2d, dst2d_e, wb)
    probs = _combine_call(inner, rpre, tpre, br2, bt2)
    return probs.reshape(EP)[:E]


# speedup vs baseline: 6.7994x; 1.0898x over previous
import functools

import jax
import jax.numpy as jnp
from jax import lax
from jax.experimental import pallas as pl
from jax.experimental.pallas import tpu as pltpu
from jax.experimental.pallas import tpu_sc as plsc

N = 50000
E = 800000
EP = 819200
CH = 128
NPAD = 50176
STRIPE = 3136
ZCH = 448
NC = 2
NS = 16
SUB = 4
B = 1000
GRID = N // B
RADIUS = 1.0

_f32 = jnp.float32
_i32 = jnp.int32


def _mesh():
    return plsc.VectorSubcoreMesh(core_axis_name="c", subcore_axis_name="s")


def _deg_body(dst_hbm, zeros_hbm, ones_hbm, out_hbm, ones_v, idx_v, acc):
    c = lax.axis_index("c")
    s = lax.axis_index("s")
    for k in range(STRIPE // ZCH):
        pltpu.sync_copy(zeros_hbm, acc.at[pl.ds(s * STRIPE + k * ZCH, ZCH)])
    plsc.subcore_barrier()
    pltpu.sync_copy(ones_hbm, ones_v)
    base0 = (c * NS + s) * (EP // (NC * NS))

    def chunk(k, carry):
        b = base0 + k * CH
        pltpu.sync_copy(dst_hbm.at[pl.ds(b, CH)], idx_v)
        pltpu.sync_copy(ones_v, acc.at[idx_v], add=True)
        return carry

    lax.fori_loop(0, EP // (NC * NS * CH), chunk, 0)
    plsc.subcore_barrier()
    pltpu.sync_copy(acc.at[pl.ds(s * STRIPE, STRIPE)],
                    out_hbm.at[pl.ds(c * NPAD + s * STRIPE, STRIPE)])


def _deg_call(dst_seg):
    f = pl.kernel(
        _deg_body,
        out_type=jax.ShapeDtypeStruct((2 * NPAD, 16), _f32),
        mesh=_mesh(),
        scratch_types=[
            pltpu.VMEM((CH, 16), _f32),
            pltpu.VMEM((CH,), _i32),
            pltpu.VMEM_SHARED((NPAD, 16), _f32),
        ],
        compiler_params=pltpu.CompilerParams(use_tc_tiling_on_sc=False),
        name="sc_degree",
    )
    return f(dst_seg, jnp.zeros((ZCH, 16), _f32), jnp.ones((CH, 16), _f32))


def _seg_body(taba_hbm, tabb_hbm, src_hbm, dst_hbm, zeros_hbm, out_hbm,
              sidx0, didx0, sidx1, didx1, rows0, rows1, acc,
              semg0, semg1, semi0, semi1, *, sub):
    c = lax.axis_index("c")
    s = lax.axis_index("s")
    for k in range(STRIPE // ZCH):
        pltpu.sync_copy(zeros_hbm, acc.at[pl.ds(s * STRIPE + k * ZCH, ZCH)])
    plsc.subcore_barrier()
    base0 = s * (EP // (NS * CH))
    idxs = [(sidx0, didx0), (sidx1, didx1)]
    rows = [rows0, rows1]
    semg = [semg0, semg1]
    semi = [semi0, semi1]

    def issue_idx(k, P):
        r0 = base0 + k * sub
        si, di = idxs[P]
        pltpu.async_copy(src_hbm.at[pl.ds(r0, sub)], si, semi[P])
        pltpu.async_copy(dst_hbm.at[pl.ds(r0, sub)], di, semi[P])

    def wait_idx(P):
        si, di = idxs[P]
        pltpu.make_async_copy(src_hbm.at[pl.ds(0, sub)], si, semi[P]).wait()
        pltpu.make_async_copy(dst_hbm.at[pl.ds(0, sub)], di, semi[P]).wait()

    def issue_g(P):
        si, _ = idxs[P]
        rw = rows[P]

        @pl.when(c == 0)
        def _():
            for i in range(sub):
                pltpu.async_copy(taba_hbm.at[si.at[i]],
                                 rw.at[pl.ds(i * CH, CH)], semg[P])

        @pl.when(c == 1)
        def _():
            for i in range(sub):
                pltpu.async_copy(tabb_hbm.at[si.at[i]],
                                 rw.at[pl.ds(i * CH, CH)], semg[P])

    def wait_g(P):
        si, _ = idxs[P]
        rw = rows[P]
        for i in range(sub):
            pltpu.make_async_copy(taba_hbm.at[si.at[i]],
                                  rw.at[pl.ds(i * CH, CH)], semg[P]).wait()

    def scatter(P):
        _, di = idxs[P]
        rw = rows[P]
        for i in range(sub):
            pltpu.sync_copy(rw.at[pl.ds(i * CH, CH)], acc.at[di.at[i]],
                            add=True)

    issue_idx(0, 0)
    wait_idx(0)
    issue_g(0)
    issue_idx(1, 1)

    def body(i, carry):
        k = 2 * i
        wait_idx(1)
        issue_g(1)
        wait_g(0)
        scatter(0)
        issue_idx(k + 2, 0)
        wait_idx(0)
        issue_g(0)
        wait_g(1)
        scatter(1)
        issue_idx(k + 3, 1)
        return carry

    lax.fori_loop(0, EP // (NS * CH * sub * 2), body, 0)
    wait_g(0)
    wait_idx(1)
    plsc.subcore_barrier()
    pltpu.sync_copy(acc.at[pl.ds(s * STRIPE, STRIPE)],
                    out_hbm.at[pl.ds(c * NPAD + s * STRIPE, STRIPE)])


def _seg_call(tab2, src2d, dst2d, w2, sub):
    f = pl.kernel(
        functools.partial(_seg_body, sub=sub),
        out_type=jax.ShapeDtypeStruct((2 * NPAD, w2), _f32),
        mesh=_mesh(),
        scratch_types=[
            pltpu.VMEM((sub, CH), _i32),
            pltpu.VMEM((sub, CH), _i32),
            pltpu.VMEM((sub, CH), _i32),
            pltpu.VMEM((sub, CH), _i32),
            pltpu.VMEM((sub * CH, w2), _f32),
            pltpu.VMEM((sub * CH, w2), _f32),
            pltpu.VMEM_SHARED((NPAD, w2), _f32),
            pltpu.SemaphoreType.DMA,
            pltpu.SemaphoreType.DMA,
            pltpu.SemaphoreType.DMA,
            pltpu.SemaphoreType.DMA,
        ],
        compiler_params=pltpu.CompilerParams(use_tc_tiling_on_sc=False),
        name=f"sc_segsum{w2}",
    )
    return f(tab2[0], tab2[1], src2d, dst2d, jnp.zeros((ZCH, w2), _f32))


ESUB = 2
ECH = ESUB * CH
ENCH = EP // (NC * NS * ECH)


def _edge_body(us_hbm, vd_hbm, src_hbm, dst_hbm, wb_hbm,
               r_hbm, t_hbm, in_hbm,
               sidx0, didx0, sidx1, didx1, pus0, pvd0, pus1, pvd1,
               wb, rbuf, tbuf, ibuf, semg0, semg1, semi0, semi1):
    c = lax.axis_index("c")
    s = lax.axis_index("s")
    pltpu.sync_copy(wb_hbm, wb)
    base0 = (c * NS + s) * (EP // (NC * NS * CH))
    lanes = lax.iota(_i32, 16)
    idxs = [(sidx0, didx0), (sidx1, didx1)]
    bufs = [(pus0, pvd0), (pus1, pvd1)]
    semg = [semg0, semg1]
    semi = [semi0, semi1]

    def col(j):
        return jnp.full((16,), j, _i32)

    def unpk(x):
        return plsc.unpack(plsc.bitcast(x, jnp.bfloat16),
                           format=plsc.PackFormat.INTERLEAVED)

    def issue_idx(k, P):
        r0 = base0 + k * ESUB
        si, di = idxs[P]
        pltpu.async_copy(src_hbm.at[pl.ds(r0, ESUB)], si, semi[P])
        pltpu.async_copy(dst_hbm.at[pl.ds(r0, ESUB)], di, semi[P])

    def wait_idx(P):
        si, di = idxs[P]
        pltpu.make_async_copy(src_hbm.at[pl.ds(0, ESUB)], si, semi[P]).wait()
        pltpu.make_async_copy(dst_hbm.at[pl.ds(0, ESUB)], di, semi[P]).wait()

    def issue_g(P):
        si, di = idxs[P]
        us, vd = bufs[P]
        for i in range(ESUB):
            pltpu.async_copy(us_hbm.at[si.at[i]],
                             us.at[pl.ds(i * CH, CH)], semg[P])
            pltpu.async_copy(vd_hbm.at[di.at[i]],
                             vd.at[pl.ds(i * CH, CH)], semg[P])

    def wait_g(P):
        si, di = idxs[P]
        us, vd = bufs[P]
        for i in range(ESUB):
            pltpu.make_async_copy(us_hbm.at[si.at[i]],
                                  us.at[pl.ds(i * CH, CH)], semg[P]).wait()
            pltpu.make_async_copy(vd_hbm.at[di.at[i]],
                                  vd.at[pl.ds(i * CH, CH)], semg[P]).wait()

    def compute(k, P):
        us, vd = bufs[P]
        b = (base0 + k * ESUB) * CH

        def group(g, carry2):
            row = g * 16 + lanes
            accr = jnp.zeros((16,), _f32)
            acct = jnp.zeros((16,), _f32)
            acci = jnp.zeros((16,), _f32)
            for w in range(64):
                xu = plsc.load_gather(us, [row, col(w)])
                xv = plsc.load_gather(vd, [row, col(w)])
                ua, ub = unpk(xu)
                va, vb = unpk(xv)
                a = ua + va
                bb = ub + vb
                acc = accr if w < 32 else acct
                acc = acc + jnp.maximum(a, 0.2 * a) * wb[2 * w]
                acc = acc + jnp.maximum(bb, 0.2 * bb) * wb[2 * w + 1]
                if w < 32:
                    accr = acc
                else:
                    acct = acc
            for j in range(17):
                a = plsc.bitcast(plsc.load_gather(us, [row, col(64 + j)]), _f32)
                bb = plsc.bitcast(plsc.load_gather(vd, [row, col(64 + j)]), _f32)
                acci = acci + a * bb
            rbuf[pl.ds(g * 16, 16)] = accr
            tbuf[pl.ds(g * 16, 16)] = acct
            ibuf[pl.ds(g * 16, 16)] = acci
            return carry2

        lax.fori_loop(0, ECH // 16, group, 0)
        pltpu.sync_copy(rbuf, r_hbm.at[pl.ds(b, ECH)])
        pltpu.sync_copy(tbuf, t_hbm.at[pl.ds(b, ECH)])
        pltpu.sync_copy(ibuf, in_hbm.at[pl.ds(b, ECH)])

    issue_idx(0, 0)
    wait_idx(0)
    issue_g(0)
    issue_idx(1, 1)

    def body(i, carry):
        k = 2 * i
        wait_idx(1)
        issue_g(1)
        wait_g(0)
        issue_idx(k + 2, 0)
        compute(k, 0)
        wait_idx(0)
        issue_g(0)
        wait_g(1)
        issue_idx(k + 3, 1)
        compute(k + 1, 1)
        return carry

    lax.fori_loop(0, ENCH // 2, body, 0)
    wait_g(0)
    wait_idx(1)


def _edge_call(US, VD, src2d, dst2d, wb):
    f = pl.kernel(
        _edge_body,
        out_type=[jax.ShapeDtypeStruct((EP,), _f32)] * 3,
        mesh=_mesh(),
        scratch_types=[
            pltpu.VMEM((ESUB, CH), _i32),
            pltpu.VMEM((ESUB, CH), _i32),
            pltpu.VMEM((ESUB, CH), _i32),
            pltpu.VMEM((ESUB, CH), _i32),
            pltpu.VMEM((ECH, 96), _i32),
            pltpu.VMEM((ECH, 96), _i32),
            pltpu.VMEM((ECH, 96), _i32),
            pltpu.VMEM((ECH, 96), _i32),
            pltpu.VMEM((128, 16), _f32),
            pltpu.VMEM((ECH,), _f32),
            pltpu.VMEM((ECH,), _f32),
            pltpu.VMEM((ECH,), _f32),
            pltpu.SemaphoreType.DMA,
            pltpu.SemaphoreType.DMA,
            pltpu.SemaphoreType.DMA,
            pltpu.SemaphoreType.DMA,
        ],
        compiler_params=pltpu.CompilerParams(
            use_tc_tiling_on_sc=False, needs_layout_passes=False),
        name="sc_edge",
    )
    return f(US, VD, src2d, dst2d, wb)


def _prep_body(z_ref, d0_ref, d1_ref, w0_ref, yp_ref, dinv_ref, zs_ref, zd_ref):
    z = z_ref[...]
    deg = d0_ref[0][:, :1] + d1_ref[0][:, :1] + 1.0
    dinv = lax.rsqrt(deg)
    dinv_ref[...] = dinv
    x0 = z[:, :1]
    alpha = jnp.maximum(x0 / RADIUS, 1.0 + 1e-7)
    acosh = jnp.log(alpha + jnp.sqrt(alpha * alpha - 1.0))
    coef = acosh / jnp.sqrt(alpha * alpha - 1.0)
    zmu = coef * jnp.concatenate([x0 - alpha * RADIUS, z[:, 1:]], axis=1)
    y = (zmu @ w0_ref[...]) * dinv
    yp_ref[0] = y[:, :32]
    yp_ref[1] = y[:, 32:]
    pad = jnp.zeros((z.shape[0], 15), _f32)
    zs_ref[...] = jnp.concatenate([-x0, z[:, 1:], pad], axis=1)
    zd_ref[...] = jnp.concatenate([x0, z[:, 1:], pad], axis=1)


def _prep_call(z, deg2, W0):
    return pl.pallas_call(
        _prep_body,
        grid=(GRID,),
        in_specs=[
            pl.BlockSpec((B, 17), lambda i: (i, 0)),
            pl.BlockSpec((1, B, 16), lambda i: (0, i, 0)),
            pl.BlockSpec((1, B, 16), lambda i: (1, i, 0)),
            pl.BlockSpec((17, 64), lambda i: (0, 0)),
        ],
        out_specs=[
            pl.BlockSpec((2, B, 32), lambda i: (0, i, 0)),
            pl.BlockSpec((B, 1), lambda i: (i, 0)),
            pl.BlockSpec((B, 32), lambda i: (i, 0)),
            pl.BlockSpec((B, 32), lambda i: (i, 0)),
        ],
        out_shape=[
            jax.ShapeDtypeStruct((2, N, 32), _f32),
            jax.ShapeDtypeStruct((N, 1), _f32),
            jax.ShapeDtypeStruct((N, 32), _f32),
            jax.ShapeDtypeStruct((N, 32), _f32),
        ],
        name="tc_prep",
    )(z, deg2, deg2, W0)


def _layer_body(acc_ref, yp_ref, dinv_ref, b_ref, w_ref, out_ref, *, relu, wo2):
    y = jnp.concatenate([acc_ref[0] + yp_ref[0], acc_ref[1] + yp_ref[1]], axis=1)
    h = dinv_ref[...] * y + b_ref[...]
    if relu:
        h = jnp.maximum(h, 0.0)
    yn = (h @ w_ref[...]) * dinv_ref[...]
    out_ref[0] = yn[:, :wo2]
    out_ref[1] = yn[:, wo2:]


def _layer_call(acc, yp, dinv, bias, W, relu):
    wi2 = yp.shape[2]
    wo2 = W.shape[1] // 2
    body = functools.partial(_layer_body, relu=relu, wo2=wo2)
    return pl.pallas_call(
        body,
        grid=(GRID,),
        in_specs=[
            pl.BlockSpec((2, B, wi2), lambda i: (0, i, 0)),
            pl.BlockSpec((2, B, wi2), lambda i: (0, i, 0)),
            pl.BlockSpec((B, 1), lambda i: (i, 0)),
            pl.BlockSpec((2 * wi2,), lambda i: (0,)),
            pl.BlockSpec(W.shape, lambda i: (0, 0)),
        ],
        out_specs=pl.BlockSpec((2, B, wo2), lambda i: (0, i, 0)),
        out_shape=jax.ShapeDtypeStruct((2, N, wo2), _f32),
        name=f"tc_layer{W.shape[1]}",
    )(acc, yp, dinv, bias, W)


def _final_node_body(acc_ref, yp_ref, dinv_ref, b_ref, wu_ref, wv_ref, bv_ref,
                     u_ref, v_ref):
    y = jnp.concatenate([acc_ref[0] + yp_ref[0], acc_ref[1] + yp_ref[1]], axis=1)
    g = dinv_ref[...] * y + b_ref[...]
    u_ref[...] = (g @ wu_ref[...]).astype(jnp.bfloat16)
    v_ref[...] = (g @ wv_ref[...] + bv_ref[...]).astype(jnp.bfloat16)


def _final_node_call(acc, yp, dinv, b2, WU, WV, bV):
    return pl.pallas_call(
        _final_node_body,
        grid=(GRID,),
        in_specs=[
            pl.BlockSpec((2, B, 16), lambda i: (0, i, 0)),
            pl.BlockSpec((2, B, 16), lambda i: (0, i, 0)),
            pl.BlockSpec((B, 1), lambda i: (i, 0)),
            pl.BlockSpec((32,), lambda i: (0,)),
            pl.BlockSpec((32, 128), lambda i: (0, 0)),
            pl.BlockSpec((32, 128), lambda i: (0, 0)),
            pl.BlockSpec((128,), lambda i: (0,)),
        ],
        out_specs=[
            pl.BlockSpec((B, 128), lambda i: (i, 0)),
            pl.BlockSpec((B, 128), lambda i: (i, 0)),
        ],
        out_shape=[
            jax.ShapeDtypeStruct((N, 128), jnp.bfloat16),
            jax.ShapeDtypeStruct((N, 128), jnp.bfloat16),
        ],
        name="tc_final_node",
    )(acc, yp, dinv, b2, WU, WV, bV)


def _combine_body(in_ref, r_ref, t_ref, br_ref, bt_ref, out_ref):
    arg = jnp.maximum(-in_ref[...] / (RADIUS * RADIUS), 1.0 + 1e-7)
    dist = -RADIUS * jnp.log(arg + jnp.sqrt(arg * arg - 1.0))
    r = r_ref[...] + br_ref[0]
    t = t_ref[...] + bt_ref[0]
    x = (dist - r) / t
    out_ref[...] = 1.0 / (1.0 + jnp.exp(-x))


def _combine_call(inner, rpre, tpre, br2, bt2):
    rows = EP // 128
    blk = 64
    return pl.pallas_call(
        _combine_body,
        grid=(rows // blk,),
        in_specs=[
            pl.BlockSpec((blk, 128), lambda i: (i, 0)),
            pl.BlockSpec((blk, 128), lambda i: (i, 0)),
            pl.BlockSpec((blk, 128), lambda i: (i, 0)),
            pl.BlockSpec(memory_space=pltpu.SMEM),
            pl.BlockSpec(memory_space=pltpu.SMEM),
        ],
        out_specs=pl.BlockSpec((blk, 128), lambda i: (i, 0)),
        out_shape=jax.ShapeDtypeStruct((rows, 128), _f32),
        name="tc_combine",
    )(inner.reshape(rows, 128), rpre.reshape(rows, 128),
      tpre.reshape(rows, 128), br2, bt2)


def kernel(z, edge_index, W0, b0, W1, b1, W2, b2,
           Wr1, br1, Wr2, br2, Wt1, bt1, Wt2, bt2):
    src = edge_index[0]
    dst = edge_index[1]
    padn = EP - E
    zero_pad = jnp.zeros((padn,), _i32)
    src_p = jnp.concatenate([src, zero_pad])
    dst_e = jnp.concatenate([dst, zero_pad])
    dst_seg = jnp.concatenate([dst, jnp.full((padn,), N, _i32)])

    extra = jnp.zeros((1024,), _i32)
    src2d = jnp.concatenate([src_p, extra]).reshape(-1, CH)
    dst2d_seg = jnp.concatenate([dst_seg, extra]).reshape(-1, CH)
    dst2d_e = jnp.concatenate([dst_e, extra]).reshape(-1, CH)

    deg2 = _deg_call(dst_seg).reshape(2, NPAD, 16)
    yp0, dinv, Zs, Zd = _prep_call(z, deg2, W0)

    acc0 = _seg_call(yp0, src2d, dst2d_seg, 32, 2).reshape(2, NPAD, 32)
    yp1 = _layer_call(acc0, yp0, dinv, b0, W1, relu=True)
    acc1 = _seg_call(yp1, src2d, dst2d_seg, 32, 2).reshape(2, NPAD, 32)
    yp2 = _layer_call(acc1, yp1, dinv, b1, W2, relu=True)
    acc2 = _seg_call(yp2, src2d, dst2d_seg, 16, 4).reshape(2, NPAD, 16)

    WU = jnp.concatenate([Wr1[:32], Wt1[:32]], axis=1)
    WV = jnp.concatenate([Wr1[32:], Wt1[32:]], axis=1)
    bV = jnp.concatenate([br1, bt1])
    U, V = _final_node_call(acc2, yp2, dinv, b2, WU, WV, bV)
    U32 = lax.bitcast_convert_type(U.reshape(N, 64, 2), _i32)
    V32 = lax.bitcast_convert_type(V.reshape(N, 64, 2), _i32)
    US = jnp.concatenate([U32, lax.bitcast_convert_type(Zs, _i32)], axis=1)
    VD = jnp.concatenate([V32, lax.bitcast_convert_type(Zd, _i32)], axis=1)

    wb = jnp.concatenate([jnp.broadcast_to(Wr2, (64, 16)),
                          jnp.broadcast_to(Wt2, (64, 16))], axis=0)
    rpre, tpre, inner = _edge_call(US, VD, src2d, dst2d_e, wb)
    probs = _combine_call(inner, rpre, tpre, br2, bt2)
    return probs.reshape(EP)[:E]
